# baseline (speedup 1.0000x reference)
import functools

import jax
import jax.numpy as jnp
from jax import lax
from jax.experimental import pallas as pl
from jax.experimental.pallas import tpu as pltpu
from jax.experimental.pallas import tpu_sc as plsc

BS, Q, C1, T = 64, 1000, 1001, 100
NUM_CLASSES = C1 - 1

BB = 4

NC, NS, L = 2, 16, 16
NW = NC * NS
B_PER_W = BS // NW
TP = 104


def _loss_kernel(tl_ref, logits_ref, out_ref):
    g = pl.program_id(0)

    s_lse = 0.0
    s_g = 0.0
    for i in range(BB):
        x = logits_ref[i]
        m = jnp.max(x, axis=-1)
        lse = m + jnp.log(jnp.sum(jnp.exp(x - m[:, None]), axis=-1))
        s_lse += jnp.sum(lse)

        cn = x[:, NUM_CLASSES:NUM_CLASSES + 1]
        unmatched = jax.lax.broadcasted_iota(jnp.int32, (Q, 1), 0) >= T
        s_g += jnp.sum(jnp.where(unmatched, cn, 0.0))

        labels = tl_ref[g * BB + i, 0]
        oh = jax.lax.broadcasted_iota(jnp.int32, (T, C1), 1) == labels[:, None]
        s_g += jnp.sum(jnp.where(oh, x[:T, :], 0.0))

    @pl.when(g == 0)
    def _():
        out_ref[0] = 0.0

    out_ref[0] += s_lse - s_g

    @pl.when(g == BS // BB - 1)
    def _():
        out_ref[0] = out_ref[0] / (BS * Q)


@functools.partial(
    pl.kernel,
    out_type=jax.ShapeDtypeStruct((NW * L,), jnp.float32),
    mesh=plsc.VectorSubcoreMesh(core_axis_name="c", subcore_axis_name="s"),
    compiler_params=pltpu.CompilerParams(needs_layout_passes=False, use_tc_tiling_on_sc=True),
    scratch_types=[
        pltpu.VMEM((TP, 4), jnp.float32),
        pltpu.VMEM((T, 4), jnp.float32),
        pltpu.VMEM((L,), jnp.float32),
    ],
)
def _sc_boxes(pb_hbm, tb_hbm, out_hbm, pb_v, tb_v, acc_v):
    wid = lax.axis_index("s") * NC + lax.axis_index("c")
    acc = jnp.zeros((L,), jnp.float32)
    for k in range(B_PER_W):
        b = wid * B_PER_W + k
        pltpu.sync_copy(pb_hbm.at[b, pl.ds(0, TP)], pb_v)
        pltpu.sync_copy(tb_hbm.at[b], tb_v)
        for i in range(T * 4 // L):
            lin = lax.iota(jnp.int32, L) + i * L
            rows = lax.shift_right_logical(lin, 2)
            cols = lax.bitwise_and(lin, 3)
            a = plsc.load_gather(pb_v, [rows, cols])
            t = plsc.load_gather(tb_v, [rows, cols])
            acc += jnp.abs(a - t)
    acc_v[...] = acc
    pltpu.sync_copy(acc_v, out_hbm.at[pl.ds(wid * L, L)])


def kernel(pred_logits, pred_boxes, tgt_boxes, tgt_labels):
    tl3 = tgt_labels.astype(jnp.int32).reshape(BS, 1, T)

    sc_part = _sc_boxes(pred_boxes, tgt_boxes)

    tc_part = pl.pallas_call(
        _loss_kernel,
        grid=(BS // BB,),
        in_specs=[
            pl.BlockSpec((BS, 1, T), lambda b: (0, 0, 0)),
            pl.BlockSpec((BB, Q, C1), lambda b: (b, 0, 0)),
        ],
        out_specs=pl.BlockSpec(memory_space=pltpu.SMEM),
        out_shape=jax.ShapeDtypeStruct((1,), jnp.float32),
    )(tl3, pred_logits)

    boxes_loss = jnp.sum(sc_part) / (BS * T * 4)
    return jnp.stack([tc_part[0], boxes_loss])


# speedup vs baseline: 1.1731x; 1.1731x over previous
import jax
import jax.numpy as jnp
from jax.experimental import pallas as pl
from jax.experimental.pallas import tpu as pltpu

BS, Q, C1, T = 64, 1000, 1001, 100
NUM_CLASSES = C1 - 1

BB = 4
TP = 104


def _loss_kernel(tl_ref, logits_ref, pb_ref, tb_ref, out_ref):
    g = pl.program_id(0)

    s_lse = 0.0
    s_g = 0.0
    l1 = 0.0
    for i in range(BB):
        x = logits_ref[i]
        m = jnp.max(x, axis=-1)
        lse = m + jnp.log(jnp.sum(jnp.exp(x - m[:, None]), axis=-1))
        s_lse += jnp.sum(lse)

        cn = x[:, NUM_CLASSES:NUM_CLASSES + 1]
        unmatched = jax.lax.broadcasted_iota(jnp.int32, (Q, 1), 0) >= T
        s_g += jnp.sum(jnp.where(unmatched, cn, 0.0))

        labels = tl_ref[g * BB + i, 0]
        oh = jax.lax.broadcasted_iota(jnp.int32, (T, C1), 1) == labels[:, None]
        s_g += jnp.sum(jnp.where(oh, x[:T, :], 0.0))

        l1 += jnp.sum(jnp.abs(pb_ref[i, :T, :] - tb_ref[i]))

    @pl.when(g == 0)
    def _():
        out_ref[0] = 0.0
        out_ref[1] = 0.0

    out_ref[0] += s_lse - s_g
    out_ref[1] += l1

    @pl.when(g == BS // BB - 1)
    def _():
        out_ref[0] = out_ref[0] / (BS * Q)
        out_ref[1] = out_ref[1] / (BS * T * 4)


def kernel(pred_logits, pred_boxes, tgt_boxes, tgt_labels):
    tl3 = tgt_labels.astype(jnp.int32).reshape(BS, 1, T)
    out = pl.pallas_call(
        _loss_kernel,
        grid=(BS // BB,),
        in_specs=[
            pl.BlockSpec((BS, 1, T), lambda b: (0, 0, 0)),
            pl.BlockSpec((BB, Q, C1), lambda b: (b, 0, 0)),
            pl.BlockSpec((BB, TP, 4), lambda b: (b, 0, 0)),
            pl.BlockSpec((BB, T, 4), lambda b: (b, 0, 0)),
        ],
        out_specs=pl.BlockSpec(memory_space=pltpu.SMEM),
        out_shape=jax.ShapeDtypeStruct((2,), jnp.float32),
    )(tl3, pred_logits, pred_boxes, tgt_boxes)
    return out
